# baseline (speedup 1.0000x reference)
import functools

import jax
import jax.numpy as jnp
from jax import lax
from jax.experimental import pallas as pl
from jax.experimental.pallas import tpu as pltpu
from jax.experimental.pallas import tpu_sc as plsc

N = 10000
E = 320000
B = 16
C = 128
COUT = 10
K = 4
HIDDEN_LAYERS = 2
EPS = 1e-5
BN_ROWS = B * N

NSPLIT = 5008
ACC_ROWS = 5120
EB = 32
EPC = E
NTILES = 16
NCORES = 2
LN = 16

RB = 640
PB = 400


def _make_lmul(combine: bool):
    mesh = plsc.VectorSubcoreMesh(core_axis_name="c", subcore_axis_name="s")

    @functools.partial(
        pl.kernel,
        mesh=mesh,
        out_type=jax.ShapeDtypeStruct((BN_ROWS, C), jnp.float32),
        scratch_types=[
            pltpu.VMEM((EB,), jnp.int32),
            pltpu.VMEM((EB,), jnp.int32),
            pltpu.VMEM((EB + LN,), jnp.float32),
            pltpu.VMEM((EB, C), jnp.float32),
            pltpu.VMEM((16, C), jnp.float32),
            pltpu.VMEM((16, C), jnp.float32),
            pltpu.VMEM((16, C), jnp.float32),
            pltpu.VMEM((32,), jnp.int32),
            pltpu.VMEM_SHARED((ACC_ROWS, C), jnp.float32),
            pltpu.SemaphoreType.DMA,
        ],
    )
    def lmul(x_ref, src_ref, dloc_ref, ew_ref, meta_ref, prev_ref, out_ref,
             idxb, dlocb, ewb, rowsb, zb, fab, fpb, metav, acc, sem):
        c = lax.axis_index("c")
        t = lax.axis_index("s")
        pltpu.sync_copy(meta_ref, metav)

        def sel(pos):
            return metav[pl.ds(pos, LN)][0]

        nb = sel(c)
        nch = sel(2 + c)
        row0 = sel(4 + c)
        b_lo = (nb * t) // NTILES
        b_hi = (nb * (t + 1)) // NTILES
        k_lo = (nch * t) // NTILES
        k_hi = (nch * (t + 1)) // NTILES

        zvec = jnp.zeros((LN,), jnp.float32)
        for r in range(16):
            for j in range(C // LN):
                zb[r, pl.ds(j * LN, LN)] = zvec

        def pass_body(pb, carry):
            xoff = pb * N
            z0 = t * (ACC_ROWS // NTILES)
            for zc in range(ACC_ROWS // NTILES // 16):
                zoff = pl.multiple_of(z0 + zc * 16, 16)
                pltpu.sync_copy(zb, acc.at[pl.ds(zoff, 16)])
            plsc.subcore_barrier()

            def edge_body(ebi, ecarry):
                off = pl.multiple_of(c * EPC + ebi * EB, EB)
                pltpu.sync_copy(src_ref.at[pl.ds(off, EB)], idxb)
                pltpu.sync_copy(dloc_ref.at[pl.ds(off, EB)], dlocb)
                pltpu.sync_copy(ew_ref.at[pl.ds(off, EB)], ewb.at[pl.ds(0, EB)])
                for j in range(EB // LN):
                    s = pl.ds(j * LN, LN)
                    idxb[s] = idxb[s] + xoff
                pltpu.async_copy(x_ref.at[idxb], rowsb, sem).wait()
                for e in range(EB):
                    w = lax.broadcast(ewb[pl.ds(e, LN)][0], (LN,))
                    for j in range(C // LN):
                        s = pl.ds(j * LN, LN)
                        rowsb[e, s] = rowsb[e, s] * w
                pltpu.sync_copy(rowsb, acc.at[dlocb], add=True)
                return ecarry

            lax.fori_loop(b_lo, b_hi, edge_body, 0)
            plsc.subcore_barrier()

            def flush_body(k, fcarry):
                koff = pl.multiple_of(k * 16, 16)
                pltpu.sync_copy(acc.at[pl.ds(koff, 16)], fab)
                orow = pl.multiple_of(xoff + row0 + k * 16, 16)
                if combine:
                    pltpu.sync_copy(prev_ref.at[pl.ds(orow, 16)], fpb)
                    for r in range(16):
                        for j in range(C // LN):
                            s = pl.ds(j * LN, LN)
                            fab[r, s] = fab[r, s] * 2.0 - fpb[r, s]
                pltpu.sync_copy(fab, out_ref.at[pl.ds(orow, 16)])
                return fcarry

            lax.fori_loop(k_lo, k_hi, flush_body, 0)
            plsc.subcore_barrier()
            return carry

        lax.fori_loop(0, B, pass_body, 0)

    return lmul


_get_lmul = functools.cache(_make_lmul)


def _transpose_kernel(x_ref, o_ref):
    o_ref[...] = jnp.transpose(x_ref[0], (1, 0))


def _transpose(x):
    return pl.pallas_call(
        _transpose_kernel,
        grid=(B,),
        in_specs=[pl.BlockSpec((1, C, N), lambda b: (b, 0, 0))],
        out_specs=pl.BlockSpec((N, C), lambda b: (b, 0)),
        out_shape=jax.ShapeDtypeStruct((BN_ROWS, C), jnp.float32),
    )(x)


def _einsum_kernel(with_stats, x0_ref, x1_ref, x2_ref, x3_ref, w_ref, b_ref,
                   o_ref, *stat_refs):
    acc = jnp.dot(x0_ref[...], w_ref[0:C], preferred_element_type=jnp.float32)
    acc += jnp.dot(x1_ref[...], w_ref[C:2 * C], preferred_element_type=jnp.float32)
    acc += jnp.dot(x2_ref[...], w_ref[2 * C:3 * C], preferred_element_type=jnp.float32)
    acc += jnp.dot(x3_ref[...], w_ref[3 * C:4 * C], preferred_element_type=jnp.float32)
    y = jnp.maximum(acc + b_ref[...], 0.0)
    o_ref[...] = y
    if with_stats:
        s_ref, q_ref = stat_refs
        ps = jnp.sum(y, axis=0, keepdims=True)
        pq = jnp.sum(y * y, axis=0, keepdims=True)
        pid = pl.program_id(0)

        @pl.when(pid == 0)
        def _init():
            s_ref[...] = ps
            q_ref[...] = pq

        @pl.when(pid != 0)
        def _acc():
            s_ref[...] += ps
            q_ref[...] += pq


def _cheb_einsum(x0, x1, x2, x3, w_flat, bias_row, with_stats):
    row_spec = pl.BlockSpec((RB, C), lambda i: (i, 0))
    full2 = pl.BlockSpec((K * C, C), lambda i: (0, 0))
    one_row = pl.BlockSpec((1, C), lambda i: (0, 0))
    out_shapes = [jax.ShapeDtypeStruct((BN_ROWS, C), jnp.float32)]
    out_specs = [row_spec]
    if with_stats:
        out_shapes += [jax.ShapeDtypeStruct((1, C), jnp.float32)] * 2
        out_specs += [one_row, one_row]
    return pl.pallas_call(
        functools.partial(_einsum_kernel, with_stats),
        grid=(BN_ROWS // RB,),
        in_specs=[row_spec, row_spec, row_spec, row_spec, full2, one_row],
        out_specs=out_specs,
        out_shape=out_shapes,
    )(x0, x1, x2, x3, w_flat, bias_row)


def _bn_kernel(h_ref, s_ref, q_ref, g_ref, b_ref, o_ref):
    m = s_ref[...] / float(BN_ROWS)
    v = q_ref[...] / float(BN_ROWS) - m * m
    scale = g_ref[...] * lax.rsqrt(v + EPS)
    shift = b_ref[...] - m * scale
    o_ref[...] = h_ref[...] * scale + shift


def _bn_apply(h, ssum, sq, gamma_row, beta_row):
    row_spec = pl.BlockSpec((RB, C), lambda i: (i, 0))
    one_row = pl.BlockSpec((1, C), lambda i: (0, 0))
    return pl.pallas_call(
        _bn_kernel,
        grid=(BN_ROWS // RB,),
        in_specs=[row_spec, one_row, one_row, one_row, one_row],
        out_specs=row_spec,
        out_shape=jax.ShapeDtypeStruct((BN_ROWS, C), jnp.float32),
    )(h, ssum, sq, gamma_row, beta_row)


def _pool_kernel(h_ref, o_ref):
    m = jnp.broadcast_to(jnp.max(h_ref[...], axis=0, keepdims=True), (8, C))
    i = pl.program_id(1)

    @pl.when(i == 0)
    def _init():
        o_ref[...] = m

    @pl.when(i != 0)
    def _acc():
        o_ref[...] = jnp.maximum(o_ref[...], m)


def _pool(h):
    return pl.pallas_call(
        _pool_kernel,
        grid=(B, N // PB),
        in_specs=[pl.BlockSpec((PB, C), lambda b, i: (b * (N // PB) + i, 0))],
        out_specs=pl.BlockSpec((8, C), lambda b, i: (b, 0)),
        out_shape=jax.ShapeDtypeStruct((B * 8, C), jnp.float32),
    )(h)


def _logsoftmax_kernel(x_ref, o_ref):
    x = x_ref[...]
    m = jnp.max(x, axis=1, keepdims=True)
    e = jnp.exp(x - m)
    s = jnp.sum(e, axis=1, keepdims=True)
    o_ref[...] = x - m - jnp.log(s)


def _log_softmax(x):
    return pl.pallas_call(
        _logsoftmax_kernel,
        out_shape=jax.ShapeDtypeStruct(x.shape, x.dtype),
    )(x)


def _prep_edges(edge_index, edge_weight):
    src = edge_index[0].astype(jnp.int32)
    dst = edge_index[1].astype(jnp.int32)
    order = jnp.argsort(dst)
    srcs = src[order]
    dsts = dst[order]
    ews = edge_weight[order]
    esplit = jnp.searchsorted(dsts, NSPLIT).astype(jnp.int32)
    starts = jnp.stack([jnp.int32(0), esplit])
    cnts = jnp.stack([esplit, jnp.int32(E) - esplit])
    i = jnp.arange(EPC, dtype=jnp.int32)
    src2, dloc2, ew2 = [], [], []
    for cc in range(NCORES):
        gidx = jnp.clip(starts[cc] + i, 0, E - 1)
        valid = i < cnts[cc]
        src2.append(jnp.where(valid, srcs[gidx], 0))
        dloc2.append(jnp.where(valid, dsts[gidx] - NSPLIT * cc, 0))
        ew2.append(jnp.where(valid, ews[gidx], 0.0))
    src2 = jnp.concatenate(src2)
    dloc2 = jnp.concatenate(dloc2)
    ew2 = jnp.concatenate(ew2)
    nb = (cnts + (EB - 1)) // EB
    meta = jnp.zeros((32,), jnp.int32)
    meta = meta.at[0].set(nb[0]).at[1].set(nb[1])
    meta = meta.at[2].set(NSPLIT // 16).at[3].set((N - NSPLIT) // 16)
    meta = meta.at[4].set(0).at[5].set(NSPLIT)
    return src2, dloc2, ew2, meta


def _cheb_layer(h, src2, dloc2, ew2, meta, w, bias, with_stats):
    x1 = _get_lmul(False)(h, src2, dloc2, ew2, meta, h)
    x2 = _get_lmul(True)(x1, src2, dloc2, ew2, meta, h)
    x3 = _get_lmul(True)(x2, src2, dloc2, ew2, meta, x1)
    w_flat = w.reshape(K * C, C)
    bias_row = bias.reshape(1, C)
    return _cheb_einsum(h, x1, x2, x3, w_flat, bias_row, with_stats)


def kernel(x, edge_index, edge_weight, W_in, b_in, gamma_h, beta_h, W_h, b_h,
           gamma_o, beta_o, W_out, b_out):
    src2, dloc2, ew2, meta = _prep_edges(edge_index, edge_weight)
    h0 = _transpose(x)

    h, s, q = _cheb_layer(h0, src2, dloc2, ew2, meta, W_in, b_in, True)
    for _ in range(HIDDEN_LAYERS):
        hn = _bn_apply(h, s, q, gamma_h.reshape(1, C), beta_h.reshape(1, C))
        h, s, q = _cheb_layer(hn, src2, dloc2, ew2, meta, W_h, b_h, True)
    hn = _bn_apply(h, s, q, gamma_o.reshape(1, C), beta_o.reshape(1, C))

    w_out_pad = jnp.pad(W_out, ((0, 0), (0, 0), (0, C - COUT)))
    b_out_pad = jnp.pad(b_out, (0, C - COUT))
    (h4,) = _cheb_layer(hn, src2, dloc2, ew2, meta, w_out_pad, b_out_pad, False)
    pooled = _pool(h4)[::8, :COUT]
    return _log_softmax(pooled)


# speedup vs baseline: 1.3971x; 1.3971x over previous
import functools

import jax
import jax.numpy as jnp
from jax import lax
from jax.experimental import pallas as pl
from jax.experimental.pallas import tpu as pltpu
from jax.experimental.pallas import tpu_sc as plsc

N = 10000
E = 320000
B = 16
C = 128
COUT = 10
K = 4
HIDDEN_LAYERS = 2
EPS = 1e-5
BN_ROWS = B * N

NSPLIT = 5008
ACC_ROWS = 5120
EB = 64
EPC = E
NTILES = 16
NCORES = 2
LN = 16

RB = 640
PB = 400


def _make_lmul(combine: bool):
    mesh = plsc.VectorSubcoreMesh(core_axis_name="c", subcore_axis_name="s")

    @functools.partial(
        pl.kernel,
        mesh=mesh,
        out_type=jax.ShapeDtypeStruct((BN_ROWS, C), jnp.float32),
        scratch_types=[
            pltpu.VMEM((EB,), jnp.int32),
            pltpu.VMEM((EB,), jnp.int32),
            pltpu.VMEM((EB, LN), jnp.float32),
            pltpu.VMEM((EB, C), jnp.float32),
            pltpu.VMEM((16, C), jnp.float32),
            pltpu.VMEM((16, C), jnp.float32),
            pltpu.VMEM((16, C), jnp.float32),
            pltpu.VMEM((32,), jnp.int32),
            pltpu.VMEM_SHARED((ACC_ROWS, C), jnp.float32),
            pltpu.SemaphoreType.DMA,
        ],
    )
    def lmul(x_ref, src_ref, dloc_ref, ew_ref, meta_ref, prev_ref, out_ref,
             idxb, dlocb, ewb, rowsb, zb, fab, fpb, metav, acc, sem):
        c = lax.axis_index("c")
        t = lax.axis_index("s")
        pltpu.sync_copy(meta_ref, metav)

        def sel(pos):
            return metav[pl.ds(pos, LN)][0]

        nb = sel(c)
        nch = sel(2 + c)
        row0 = sel(4 + c)
        b_lo = (nb * t) // NTILES
        b_hi = (nb * (t + 1)) // NTILES
        k_lo = (nch * t) // NTILES
        k_hi = (nch * (t + 1)) // NTILES

        zvec = jnp.zeros((LN,), jnp.float32)
        for r in range(16):
            for j in range(C // LN):
                zb[r, pl.ds(j * LN, LN)] = zvec

        def pass_body(pb, carry):
            xoff = pb * N
            z0 = t * (ACC_ROWS // NTILES)
            for zc in range(ACC_ROWS // NTILES // 16):
                zoff = pl.multiple_of(z0 + zc * 16, 16)
                pltpu.sync_copy(zb, acc.at[pl.ds(zoff, 16)])
            plsc.subcore_barrier()

            def edge_body(ebi, ecarry):
                off = pl.multiple_of(c * EPC + ebi * EB, EB)
                pltpu.sync_copy(src_ref.at[pl.ds(off, EB)], idxb)
                pltpu.sync_copy(dloc_ref.at[pl.ds(off, EB)], dlocb)
                pltpu.sync_copy(ew_ref.at[pl.ds(off, EB)], ewb)
                for j in range(EB // LN):
                    s = pl.ds(j * LN, LN)
                    idxb[s] = idxb[s] + xoff
                pltpu.async_copy(x_ref.at[idxb], rowsb, sem).wait()
                for e in range(EB):
                    w = ewb[e, pl.ds(0, LN)]
                    for j in range(C // LN):
                        s = pl.ds(j * LN, LN)
                        rowsb[e, s] = rowsb[e, s] * w
                pltpu.sync_copy(rowsb, acc.at[dlocb], add=True)
                return ecarry

            lax.fori_loop(b_lo, b_hi, edge_body, 0)
            plsc.subcore_barrier()

            def flush_body(k, fcarry):
                koff = pl.multiple_of(k * 16, 16)
                pltpu.sync_copy(acc.at[pl.ds(koff, 16)], fab)
                orow = pl.multiple_of(xoff + row0 + k * 16, 16)
                if combine:
                    pltpu.sync_copy(prev_ref.at[pl.ds(orow, 16)], fpb)
                    for r in range(16):
                        for j in range(C // LN):
                            s = pl.ds(j * LN, LN)
                            fab[r, s] = fab[r, s] * 2.0 - fpb[r, s]
                pltpu.sync_copy(fab, out_ref.at[pl.ds(orow, 16)])
                return fcarry

            lax.fori_loop(k_lo, k_hi, flush_body, 0)
            plsc.subcore_barrier()
            return carry

        lax.fori_loop(0, B, pass_body, 0)

    return lmul


_get_lmul = functools.cache(_make_lmul)


def _transpose_kernel(x_ref, o_ref):
    o_ref[...] = jnp.transpose(x_ref[0], (1, 0))


def _transpose(x):
    return pl.pallas_call(
        _transpose_kernel,
        grid=(B,),
        in_specs=[pl.BlockSpec((1, C, N), lambda b: (b, 0, 0))],
        out_specs=pl.BlockSpec((N, C), lambda b: (b, 0)),
        out_shape=jax.ShapeDtypeStruct((BN_ROWS, C), jnp.float32),
    )(x)


def _einsum_kernel(with_stats, x0_ref, x1_ref, x2_ref, x3_ref, w_ref, b_ref,
                   o_ref, *stat_refs):
    acc = jnp.dot(x0_ref[...], w_ref[0:C], preferred_element_type=jnp.float32)
    acc += jnp.dot(x1_ref[...], w_ref[C:2 * C], preferred_element_type=jnp.float32)
    acc += jnp.dot(x2_ref[...], w_ref[2 * C:3 * C], preferred_element_type=jnp.float32)
    acc += jnp.dot(x3_ref[...], w_ref[3 * C:4 * C], preferred_element_type=jnp.float32)
    y = jnp.maximum(acc + b_ref[...], 0.0)
    o_ref[...] = y
    if with_stats:
        s_ref, q_ref = stat_refs
        ps = jnp.sum(y, axis=0, keepdims=True)
        pq = jnp.sum(y * y, axis=0, keepdims=True)
        pid = pl.program_id(0)

        @pl.when(pid == 0)
        def _init():
            s_ref[...] = ps
            q_ref[...] = pq

        @pl.when(pid != 0)
        def _acc():
            s_ref[...] += ps
            q_ref[...] += pq


def _cheb_einsum(x0, x1, x2, x3, w_flat, bias_row, with_stats):
    row_spec = pl.BlockSpec((RB, C), lambda i: (i, 0))
    full2 = pl.BlockSpec((K * C, C), lambda i: (0, 0))
    one_row = pl.BlockSpec((1, C), lambda i: (0, 0))
    out_shapes = [jax.ShapeDtypeStruct((BN_ROWS, C), jnp.float32)]
    out_specs = [row_spec]
    if with_stats:
        out_shapes += [jax.ShapeDtypeStruct((1, C), jnp.float32)] * 2
        out_specs += [one_row, one_row]
    return pl.pallas_call(
        functools.partial(_einsum_kernel, with_stats),
        grid=(BN_ROWS // RB,),
        in_specs=[row_spec, row_spec, row_spec, row_spec, full2, one_row],
        out_specs=out_specs,
        out_shape=out_shapes,
    )(x0, x1, x2, x3, w_flat, bias_row)


def _bn_kernel(h_ref, s_ref, q_ref, g_ref, b_ref, o_ref):
    m = s_ref[...] / float(BN_ROWS)
    v = q_ref[...] / float(BN_ROWS) - m * m
    scale = g_ref[...] * lax.rsqrt(v + EPS)
    shift = b_ref[...] - m * scale
    o_ref[...] = h_ref[...] * scale + shift


def _bn_apply(h, ssum, sq, gamma_row, beta_row):
    row_spec = pl.BlockSpec((RB, C), lambda i: (i, 0))
    one_row = pl.BlockSpec((1, C), lambda i: (0, 0))
    return pl.pallas_call(
        _bn_kernel,
        grid=(BN_ROWS // RB,),
        in_specs=[row_spec, one_row, one_row, one_row, one_row],
        out_specs=row_spec,
        out_shape=jax.ShapeDtypeStruct((BN_ROWS, C), jnp.float32),
    )(h, ssum, sq, gamma_row, beta_row)


def _pool_kernel(h_ref, o_ref):
    m = jnp.broadcast_to(jnp.max(h_ref[...], axis=0, keepdims=True), (8, C))
    i = pl.program_id(1)

    @pl.when(i == 0)
    def _init():
        o_ref[...] = m

    @pl.when(i != 0)
    def _acc():
        o_ref[...] = jnp.maximum(o_ref[...], m)


def _pool(h):
    return pl.pallas_call(
        _pool_kernel,
        grid=(B, N // PB),
        in_specs=[pl.BlockSpec((PB, C), lambda b, i: (b * (N // PB) + i, 0))],
        out_specs=pl.BlockSpec((8, C), lambda b, i: (b, 0)),
        out_shape=jax.ShapeDtypeStruct((B * 8, C), jnp.float32),
    )(h)


def _logsoftmax_kernel(x_ref, o_ref):
    x = x_ref[...]
    m = jnp.max(x, axis=1, keepdims=True)
    e = jnp.exp(x - m)
    s = jnp.sum(e, axis=1, keepdims=True)
    o_ref[...] = x - m - jnp.log(s)


def _log_softmax(x):
    return pl.pallas_call(
        _logsoftmax_kernel,
        out_shape=jax.ShapeDtypeStruct(x.shape, x.dtype),
    )(x)


def _prep_edges(edge_index, edge_weight):
    src = edge_index[0].astype(jnp.int32)
    dst = edge_index[1].astype(jnp.int32)
    order = jnp.argsort(dst)
    srcs = src[order]
    dsts = dst[order]
    ews = edge_weight[order]
    esplit = jnp.searchsorted(dsts, NSPLIT).astype(jnp.int32)
    starts = jnp.stack([jnp.int32(0), esplit])
    cnts = jnp.stack([esplit, jnp.int32(E) - esplit])
    i = jnp.arange(EPC, dtype=jnp.int32)
    src2, dloc2, ew2 = [], [], []
    for cc in range(NCORES):
        gidx = jnp.clip(starts[cc] + i, 0, E - 1)
        valid = i < cnts[cc]
        src2.append(jnp.where(valid, srcs[gidx], 0))
        dloc2.append(jnp.where(valid, dsts[gidx] - NSPLIT * cc, 0))
        ew2.append(jnp.where(valid, ews[gidx], 0.0))
    ew2 = [jnp.repeat(e[:, None], LN, axis=1) for e in ew2]
    src2 = jnp.concatenate(src2)
    dloc2 = jnp.concatenate(dloc2)
    ew2 = jnp.concatenate(ew2)
    nb = (cnts + (EB - 1)) // EB
    meta = jnp.zeros((32,), jnp.int32)
    meta = meta.at[0].set(nb[0]).at[1].set(nb[1])
    meta = meta.at[2].set(NSPLIT // 16).at[3].set((N - NSPLIT) // 16)
    meta = meta.at[4].set(0).at[5].set(NSPLIT)
    return src2, dloc2, ew2, meta


def _cheb_layer(h, src2, dloc2, ew2, meta, w, bias, with_stats):
    x1 = _get_lmul(False)(h, src2, dloc2, ew2, meta, h)
    x2 = _get_lmul(True)(x1, src2, dloc2, ew2, meta, h)
    x3 = _get_lmul(True)(x2, src2, dloc2, ew2, meta, x1)
    w_flat = w.reshape(K * C, C)
    bias_row = bias.reshape(1, C)
    return _cheb_einsum(h, x1, x2, x3, w_flat, bias_row, with_stats)


def kernel(x, edge_index, edge_weight, W_in, b_in, gamma_h, beta_h, W_h, b_h,
           gamma_o, beta_o, W_out, b_out):
    src2, dloc2, ew2, meta = _prep_edges(edge_index, edge_weight)
    h0 = _transpose(x)

    h, s, q = _cheb_layer(h0, src2, dloc2, ew2, meta, W_in, b_in, True)
    for _ in range(HIDDEN_LAYERS):
        hn = _bn_apply(h, s, q, gamma_h.reshape(1, C), beta_h.reshape(1, C))
        h, s, q = _cheb_layer(hn, src2, dloc2, ew2, meta, W_h, b_h, True)
    hn = _bn_apply(h, s, q, gamma_o.reshape(1, C), beta_o.reshape(1, C))

    w_out_pad = jnp.pad(W_out, ((0, 0), (0, 0), (0, C - COUT)))
    b_out_pad = jnp.pad(b_out, (0, C - COUT))
    (h4,) = _cheb_layer(hn, src2, dloc2, ew2, meta, w_out_pad, b_out_pad, False)
    pooled = _pool(h4)[::8, :COUT]
    return _log_softmax(pooled)
